# baseline (speedup 1.0000x reference)
import functools

import jax
import jax.numpy as jnp
from jax import lax
from jax.experimental import pallas as pl
from jax.experimental.pallas import tpu as pltpu
from jax.experimental.pallas import tpu_sc as plsc

_N0, _N1, _N2 = 100000, 25000, 2500
_E0, _E1 = 400000, 40000
_F = 128
_FH = 64
_K = 128

_K0 = 112
_G0 = 8
_NG0 = 28
_NCH0 = _NG0 * _G0
_EPT0 = _NG0 * _G0 * _K0
_E0P = 16 * _EPT0
_B0 = 12504
_ACC0 = 12512
_CNT0 = 12544
_ZSP0 = 784
_GSZ = _G0 * _K0
_CAP0 = 30 * _GSZ

_NCH1 = 10
_EPT1 = _NCH1 * _K
_E1H = 16 * _EPT1
_ACC1 = 2512
_CNT1 = 2560
_ZSP1 = 160

_mesh = plsc.VectorSubcoreMesh(
    core_axis_name="c", subcore_axis_name="s", num_cores=2, num_subcores=16)


@functools.partial(
    pl.kernel,
    out_type=[
        jax.ShapeDtypeStruct((2, 16, _CNT0), jnp.float32),
        jax.ShapeDtypeStruct((2, 16, _CNT1), jnp.float32),
        jax.ShapeDtypeStruct((2, 16 * _CAP0), jnp.int32),
        jax.ShapeDtypeStruct((2, 16 * _CAP0), jnp.int32),
        jax.ShapeDtypeStruct((2, 16 * 128), jnp.int32),
    ],
    mesh=_mesh,
    compiler_params=pltpu.CompilerParams(needs_layout_passes=False),
    scratch_types=[
        pltpu.VMEM((_GSZ,), jnp.int32),
        pltpu.VMEM((_GSZ,), jnp.int32),
        pltpu.VMEM((_EPT1,), jnp.int32),
        pltpu.VMEM((_CNT0,), jnp.float32),
        pltpu.VMEM((_CNT1,), jnp.float32),
        pltpu.VMEM((_CAP0 + 16,), jnp.int32),
        pltpu.VMEM((_CAP0 + 16,), jnp.int32),
        pltpu.VMEM((128,), jnp.int32),
    ],
)
def _counts(s0_r, d0_r, d1_r, z_r, c0_r, c1_r, cs_r, cd_r, gc_r,
            s0_g, d0_g, d1_v, c0_v, c1_v, cs_v, cd_v, gc_v):
  c = lax.axis_index("c")
  s = lax.axis_index("s")
  pltpu.sync_copy(d1_r.at[c, pl.ds(s * _EPT1, _EPT1)], d1_v)
  pltpu.sync_copy(z_r, c0_v)
  pltpu.sync_copy(z_r.at[pl.ds(0, _CNT1)], c1_v)
  one16 = jnp.full((16,), 1.0, jnp.float32)

  def grp(g, ofs):
    pltpu.sync_copy(s0_r.at[pl.ds(s * _EPT0 + g * _GSZ, _GSZ)], s0_g)
    pltpu.sync_copy(d0_r.at[c, pl.ds(s * _EPT0 + g * _GSZ, _GSZ)], d0_g)

    def vec(t, ofs2):
      dstv = d0_g[pl.ds(t * 16, 16)]
      srcv = s0_g[pl.ds(t * 16, 16)]
      plsc.addupdate_scatter(c0_v, [dstv], one16)
      mask = dstv < _B0
      plsc.store_compressed(cs_v.at[pl.ds(ofs2, 16)], srcv, mask=mask)
      plsc.store_compressed(cd_v.at[pl.ds(ofs2, 16)], dstv, mask=mask)
      return ofs2 + plsc.all_reduce_population_count(mask)[0]

    return lax.fori_loop(0, _GSZ // 16, vec, ofs)

  m = lax.fori_loop(0, _NG0, grp, jnp.int32(0))

  dum_s = jnp.zeros((16,), jnp.int32)
  dum_d = jnp.full((16,), _B0, jnp.int32)

  def pad(t, carry):
    cs_v[pl.ds(m + t * 16, 16)] = dum_s
    cd_v[pl.ds(m + t * 16, 16)] = dum_d
    return carry

  lax.fori_loop(0, _GSZ // 16 + 1, pad, 0)
  ng = (m + _GSZ - 1) // _GSZ
  gc_v[pl.ds(0, 16)] = jnp.full((16,), 1, jnp.int32) * ng

  def h1(t, carry):
    plsc.addupdate_scatter(c1_v, [d1_v[pl.ds(t * 16, 16)]], one16)
    return carry

  lax.fori_loop(0, _EPT1 // 16, h1, 0)
  pltpu.sync_copy(c0_v, c0_r.at[c, s])
  pltpu.sync_copy(c1_v, c1_r.at[c, s])
  pltpu.sync_copy(cs_v.at[pl.ds(0, _CAP0)], cs_r.at[c, pl.ds(s * _CAP0, _CAP0)])
  pltpu.sync_copy(cd_v.at[pl.ds(0, _CAP0)], cd_r.at[c, pl.ds(s * _CAP0, _CAP0)])
  pltpu.sync_copy(gc_v, gc_r.at[c, pl.ds(s * 128, 128)])


def _pre(x, hist, w0, b0):
  bm = 1000

  def body(x_r, h_r, w_r, b_r, o_r):
    h = jnp.dot(x_r[...], w_r[...], preferred_element_type=jnp.float32)
    o_r[...] = jnp.maximum(h + b_r[...], 0.0) - h_r[...]

  return pl.pallas_call(
      body,
      grid=(_N0 // bm,),
      in_specs=[
          pl.BlockSpec((bm, _F), lambda i: (i, 0)),
          pl.BlockSpec((bm, _F), lambda i: (i, 0)),
          pl.BlockSpec((_F, _F), lambda i: (0, 0)),
          pl.BlockSpec((1, _F), lambda i: (0, 0)),
      ],
      out_specs=pl.BlockSpec((bm, _F), lambda i: (i, 0)),
      out_shape=jax.ShapeDtypeStruct((_N0, _F), jnp.float32),
  )(x, hist, w0, b0.reshape(1, _F))


@functools.partial(
    pl.kernel,
    out_type=jax.ShapeDtypeStruct((_N1, _F), jnp.float32),
    mesh=_mesh,
    scratch_types=[
        pltpu.VMEM((_G0, _K0), jnp.int32),
        pltpu.VMEM((_G0, _K0), jnp.int32),
        pltpu.VMEM((2, _K0, _F), jnp.float32),
        pltpu.VMEM((128,), jnp.int32),
        pltpu.VMEM_SHARED((_ACC0, _F), jnp.float32),
        pltpu.SemaphoreType.DMA,
        pltpu.SemaphoreType.DMA,
        pltpu.SemaphoreType.DMA,
        pltpu.SemaphoreType.DMA,
    ],
)
def _agg0(h_r, src_r, dsts_r, gc_r, z128_r, sums_r,
          src_g, dst_g, rows_v, gc_v, acc, ga, gb, sa, sb):
  c = lax.axis_index("c")
  s = lax.axis_index("s")
  z0 = jnp.where(s == 15, _ACC0 - _ZSP0, s * _ZSP0)
  pltpu.sync_copy(z128_r, acc.at[pl.ds(z0, _ZSP0)])
  pltpu.sync_copy(gc_r.at[c, pl.ds(s * 128, 128)], gc_v)
  plsc.subcore_barrier()

  gsem = (ga, gb)
  ssem = (sa, sb)
  ng = gc_v[pl.ds(0, 16)][0]

  def group(g, carry):
    @pl.when(g < ng)
    def _():
      pltpu.sync_copy(src_r.at[c, s, pl.ds(g * _G0, _G0)], src_g)
      pltpu.sync_copy(dsts_r.at[c, s, pl.ds(g * _G0, _G0)], dst_g)

      def gather(k, b):
        return pltpu.make_async_copy(
            h_r.at[src_g.at[k]], rows_v.at[b], gsem[b])

      def scatter(k, b):
        return pltpu.make_async_copy(
            rows_v.at[b], acc.at[dst_g.at[k]], ssem[b])

      for k in range(_G0):
        b = k & 1
        if k >= 2:
          scatter(k - 2, b).wait()
        gk = gather(k, b)
        gk.start()
        gk.wait()
        scatter(k, b).start(add=True)
      scatter(_G0 - 2, 0).wait()
      scatter(_G0 - 1, 1).wait()

    return carry

  lax.fori_loop(0, _NG0, group, 0)
  plsc.subcore_barrier()
  w0 = jnp.where(s == 15,
                 jnp.where(c == 0, _B0 - _ZSP0, (_N1 - _B0) - _ZSP0),
                 s * _ZSP0)
  pltpu.sync_copy(acc.at[pl.ds(w0, _ZSP0)],
                  sums_r.at[pl.ds(c * _B0 + w0, _ZSP0)])


def _mid(sums, cnt, agg0, hist1, w1, b1, w2):
  bm = 1000

  def body(s_r, c_r, a_r, h_r, w1_r, b1_r, w2_r, y_r):
    n = jnp.maximum(jnp.sum(c_r[...], axis=1, keepdims=True), 1.0)
    m = s_r[...] / n + a_r[...]
    t = jnp.dot(m, w1_r[...], preferred_element_type=jnp.float32) + b1_r[...]
    tcat = jnp.concatenate([t, jnp.maximum(t, 0.0)], axis=1) - h_r[...]
    y = jnp.dot(tcat, w2_r[...], preferred_element_type=jnp.float32)
    y_r[...] = jnp.concatenate(
        [y, jnp.zeros((bm, _F - _FH), jnp.float32)], axis=1)

  return pl.pallas_call(
      body,
      grid=(_N1 // bm,),
      in_specs=[
          pl.BlockSpec((bm, _F), lambda i: (i, 0)),
          pl.BlockSpec((bm, 16), lambda i: (i, 0)),
          pl.BlockSpec((bm, _F), lambda i: (i, 0)),
          pl.BlockSpec((bm, 2 * _F), lambda i: (i, 0)),
          pl.BlockSpec((_F, _F), lambda i: (0, 0)),
          pl.BlockSpec((1, _F), lambda i: (0, 0)),
          pl.BlockSpec((2 * _F, _FH), lambda i: (0, 0)),
      ],
      out_specs=pl.BlockSpec((bm, _F), lambda i: (i, 0)),
      out_shape=jax.ShapeDtypeStruct((_N1, _F), jnp.float32),
  )(sums, cnt, agg0, hist1, w1, b1.reshape(1, _F), w2)


@functools.partial(
    pl.kernel,
    out_type=jax.ShapeDtypeStruct((2, _ACC1, _F), jnp.float32),
    mesh=_mesh,
    scratch_types=[
        pltpu.VMEM((_EPT1,), jnp.int32),
        pltpu.VMEM((_NCH1, _K), jnp.int32),
        pltpu.VMEM((2, _K, _F), jnp.float32),
        pltpu.VMEM_SHARED((_ACC1, _F), jnp.float32),
        pltpu.SemaphoreType.DMA,
        pltpu.SemaphoreType.DMA,
        pltpu.SemaphoreType.DMA,
        pltpu.SemaphoreType.DMA,
    ],
)
def _agg1(y_r, srcs_r, dsts_r, z128_r, psum_r, src_v, dst_v, rows_v, acc,
          ga, gb, sa, sb):
  c = lax.axis_index("c")
  s = lax.axis_index("s")
  z0 = jnp.where(s == 15, _ACC1 - _ZSP1, s * _ZSP1)
  pltpu.sync_copy(z128_r.at[pl.ds(0, _ZSP1)], acc.at[pl.ds(z0, _ZSP1)])
  pltpu.sync_copy(srcs_r.at[c, pl.ds(s * _EPT1, _EPT1)], src_v)
  pltpu.sync_copy(dsts_r.at[c, s], dst_v)
  plsc.subcore_barrier()

  gsem = (ga, gb)
  ssem = (sa, sb)

  def gather(j, b):
    return pltpu.make_async_copy(
        y_r.at[src_v.at[pl.ds(j * _K, _K)]], rows_v.at[b], gsem[b])

  def scatter(j, b):
    return pltpu.make_async_copy(rows_v.at[b], acc.at[dst_v.at[j]], ssem[b])

  for j in range(_NCH1):
    b = j & 1
    if j >= 2:
      scatter(j - 2, b).wait()
    gj = gather(j, b)
    gj.start()
    gj.wait()
    scatter(j, b).start(add=True)
  scatter(_NCH1 - 2, 0).wait()
  scatter(_NCH1 - 1, 1).wait()
  plsc.subcore_barrier()
  pltpu.sync_copy(acc.at[pl.ds(z0, _ZSP1)], psum_r.at[c, pl.ds(z0, _ZSP1)])


def _post(psums, cnt1, agg1, w2, b2):
  def body(ps_r, c_r, a_r, w_r, b_r, o_r):
    sums = (ps_r[0] + ps_r[1])[:_N2, :_FH]
    n = jnp.maximum(jnp.sum(c_r[...], axis=1, keepdims=True), 1.0)
    z = jnp.dot(a_r[...], w_r[...], preferred_element_type=jnp.float32)
    o_r[...] = sums / n + z + b_r[...]

  return pl.pallas_call(
      body,
      grid=(1,),
      in_specs=[
          pl.BlockSpec((2, _ACC1, _F), lambda i: (0, 0, 0)),
          pl.BlockSpec((_N2, 32), lambda i: (0, 0)),
          pl.BlockSpec((_N2, 2 * _F), lambda i: (0, 0)),
          pl.BlockSpec((2 * _F, _FH), lambda i: (0, 0)),
          pl.BlockSpec((1, _FH), lambda i: (0, 0)),
      ],
      out_specs=pl.BlockSpec((_N2, _FH), lambda i: (0, 0)),
      out_shape=jax.ShapeDtypeStruct((_N2, _FH), jnp.float32),
  )(psums, cnt1, agg1, w2, b2.reshape(1, _FH))


def _pad1d(a, n, val):
  return jnp.concatenate([a, jnp.full((n - a.shape[0],), val, a.dtype)])


def kernel(preprocess, h_hist_0, h_hist_1, agg_h_0, agg_h_1,
           edge_index_0, edge_index_1, W0, b0, W1, b1, W2, b2):
  z128 = jnp.zeros((_ZSP0, _F), jnp.float32)
  z1d = jnp.zeros((_CNT0,), jnp.float32)

  src0 = _pad1d(edge_index_0[0], _E0P, 0)
  d0 = _pad1d(edge_index_0[1], _E0P, _N1)
  dsts0 = jnp.stack([
      jnp.where(d0 < _B0, d0, _B0),
      jnp.where(d0 >= _B0, d0 - _B0, _B0),
  ])
  half = _E1 // 2
  srcs1 = jnp.stack([
      _pad1d(edge_index_1[0, :half], _E1H, 0),
      _pad1d(edge_index_1[0, half:], _E1H, 0),
  ])
  dsts1 = jnp.stack([
      _pad1d(edge_index_1[1, :half], _E1H, _N2),
      _pad1d(edge_index_1[1, half:], _E1H, _N2),
  ]).reshape(2, 16, _NCH1, _K)

  cnt0, cnt1, csrc, cdst, gcnt = _counts(
      src0, dsts0, dsts1.reshape(2, _E1H), z1d)
  cnt0_t = jnp.concatenate(
      [cnt0[0].T[:_B0], cnt0[1].T[:_N1 - _B0]], axis=0)
  cnt1_t = jnp.concatenate([cnt1[0].T[:_N2], cnt1[1].T[:_N2]], axis=1)

  h = _pre(preprocess, h_hist_0, W0, b0)

  nch = _CAP0 // _K0
  sums0 = _agg0(h, csrc.reshape(2, 16, nch, _K0),
                cdst.reshape(2, 16, nch, _K0), gcnt, z128)

  y = _mid(sums0, cnt0_t, agg_h_0, h_hist_1, W1, b1, W2)

  psums = _agg1(y, srcs1, dsts1, z128)

  return _post(psums, cnt1_t, agg_h_1, W2, b2)


# speedup vs baseline: 1.4658x; 1.4658x over previous
import functools

import jax
import jax.numpy as jnp
from jax import lax
from jax.experimental import pallas as pl
from jax.experimental.pallas import tpu as pltpu
from jax.experimental.pallas import tpu_sc as plsc

_N0, _N1, _N2 = 100000, 25000, 2500
_E0, _E1 = 400000, 40000
_F = 128
_FH = 64
_K = 128

_K0 = 112
_G0 = 8
_NG0 = 28
_NCH0 = _NG0 * _G0
_EPT0 = _NCH0 * _K0
_E0P = 16 * _EPT0
_B0 = 12504
_ACC0 = 12512
_CNT0 = 12544
_ZSP0 = 784
_GSZ = _G0 * _K0

_NCH1 = 10
_EPT1 = _NCH1 * _K
_E1H = 16 * _EPT1
_ACC1 = 2512
_CNT1 = 2560
_ZSP1 = 160

_mesh = plsc.VectorSubcoreMesh(
    core_axis_name="c", subcore_axis_name="s", num_cores=2, num_subcores=16)


@functools.partial(
    pl.kernel,
    out_type=[
        jax.ShapeDtypeStruct((2, 16, _CNT0), jnp.float32),
        jax.ShapeDtypeStruct((2, 16, _CNT1), jnp.float32),
    ],
    mesh=_mesh,
    compiler_params=pltpu.CompilerParams(needs_layout_passes=False),
    scratch_types=[
        pltpu.VMEM((_NCH0, _K0), jnp.int32),
        pltpu.VMEM((_NCH1, _K), jnp.int32),
        pltpu.VMEM((_CNT0,), jnp.float32),
        pltpu.VMEM((_CNT1,), jnp.float32),
    ],
)
def _counts(d0_r, d1_r, z_r, c0_r, c1_r, d0_v, d1_v, c0_v, c1_v):
  c = lax.axis_index("c")
  s = lax.axis_index("s")
  pltpu.sync_copy(d0_r.at[c, s], d0_v)
  pltpu.sync_copy(d1_r.at[c, s], d1_v)
  pltpu.sync_copy(z_r, c0_v)
  pltpu.sync_copy(z_r.at[pl.ds(0, _CNT1)], c1_v)
  one16 = jnp.full((16,), 1.0, jnp.float32)

  def h0(j, carry):
    def inner(t, carry2):
      plsc.addupdate_scatter(c0_v, [d0_v[j, pl.ds(t * 16, 16)]], one16)
      return carry2
    return lax.fori_loop(0, _K0 // 16, inner, carry)

  lax.fori_loop(0, _NCH0, h0, 0)

  def h1(j, carry):
    def inner(t, carry2):
      plsc.addupdate_scatter(c1_v, [d1_v[j, pl.ds(t * 16, 16)]], one16)
      return carry2
    return lax.fori_loop(0, _K // 16, inner, carry)

  lax.fori_loop(0, _NCH1, h1, 0)
  pltpu.sync_copy(c0_v, c0_r.at[c, s])
  pltpu.sync_copy(c1_v, c1_r.at[c, s])


def _pre(x, hist, w0, b0):
  bm = 1000

  def body(x_r, h_r, w_r, b_r, o_r):
    h = jnp.dot(x_r[...], w_r[...], preferred_element_type=jnp.float32)
    o_r[...] = jnp.maximum(h + b_r[...], 0.0) - h_r[...]

  return pl.pallas_call(
      body,
      grid=(_N0 // bm,),
      in_specs=[
          pl.BlockSpec((bm, _F), lambda i: (i, 0)),
          pl.BlockSpec((bm, _F), lambda i: (i, 0)),
          pl.BlockSpec((_F, _F), lambda i: (0, 0)),
          pl.BlockSpec((1, _F), lambda i: (0, 0)),
      ],
      out_specs=pl.BlockSpec((bm, _F), lambda i: (i, 0)),
      out_shape=jax.ShapeDtypeStruct((_N0, _F), jnp.float32),
  )(x, hist, w0, b0.reshape(1, _F))


@functools.partial(
    pl.kernel,
    out_type=jax.ShapeDtypeStruct((_N1, _F), jnp.float32),
    mesh=_mesh,
    scratch_types=[
        pltpu.VMEM((_GSZ,), jnp.int32),
        pltpu.VMEM((_G0, _K0), jnp.int32),
        pltpu.VMEM((2, _K0, _F), jnp.float32),
        pltpu.VMEM_SHARED((_ACC0, _F), jnp.float32),
        pltpu.SemaphoreType.DMA,
        pltpu.SemaphoreType.DMA,
        pltpu.SemaphoreType.DMA,
        pltpu.SemaphoreType.DMA,
    ],
)
def _agg0(h_r, src_r, dsts_r, z128_r, sums_r,
          src_g, dst_g, rows_v, acc, ga, gb, sa, sb):
  c = lax.axis_index("c")
  s = lax.axis_index("s")
  z0 = jnp.where(s == 15, _ACC0 - _ZSP0, s * _ZSP0)
  pltpu.sync_copy(z128_r, acc.at[pl.ds(z0, _ZSP0)])
  plsc.subcore_barrier()

  gsem = (ga, gb)
  ssem = (sa, sb)

  def group(g, carry):
    pltpu.sync_copy(src_r.at[pl.ds(s * _EPT0 + g * _GSZ, _GSZ)], src_g)
    pltpu.sync_copy(dsts_r.at[c, s, pl.ds(g * _G0, _G0)], dst_g)

    def gather(k, b):
      return pltpu.make_async_copy(
          h_r.at[src_g.at[pl.ds(k * _K0, _K0)]], rows_v.at[b], gsem[b])

    def scatter(k, b):
      return pltpu.make_async_copy(rows_v.at[b], acc.at[dst_g.at[k]], ssem[b])

    for k in range(_G0):
      b = k & 1
      if k >= 2:
        scatter(k - 2, b).wait()
      gk = gather(k, b)
      gk.start()
      gk.wait()
      scatter(k, b).start(add=True)
    scatter(_G0 - 2, 0).wait()
    scatter(_G0 - 1, 1).wait()
    return carry

  lax.fori_loop(0, _NG0, group, 0)
  plsc.subcore_barrier()
  w0 = jnp.where(s == 15,
                 jnp.where(c == 0, _B0 - _ZSP0, (_N1 - _B0) - _ZSP0),
                 s * _ZSP0)
  pltpu.sync_copy(acc.at[pl.ds(w0, _ZSP0)],
                  sums_r.at[pl.ds(c * _B0 + w0, _ZSP0)])


def _mid(sums, cnt, agg0, hist1, w1, b1, w2):
  bm = 1000

  def body(s_r, c_r, a_r, h_r, w1_r, b1_r, w2_r, y_r):
    n = jnp.maximum(jnp.sum(c_r[...], axis=1, keepdims=True), 1.0)
    m = s_r[...] / n + a_r[...]
    t = jnp.dot(m, w1_r[...], preferred_element_type=jnp.float32) + b1_r[...]
    tcat = jnp.concatenate([t, jnp.maximum(t, 0.0)], axis=1) - h_r[...]
    y = jnp.dot(tcat, w2_r[...], preferred_element_type=jnp.float32)
    y_r[...] = jnp.concatenate(
        [y, jnp.zeros((bm, _F - _FH), jnp.float32)], axis=1)

  return pl.pallas_call(
      body,
      grid=(_N1 // bm,),
      in_specs=[
          pl.BlockSpec((bm, _F), lambda i: (i, 0)),
          pl.BlockSpec((bm, 16), lambda i: (i, 0)),
          pl.BlockSpec((bm, _F), lambda i: (i, 0)),
          pl.BlockSpec((bm, 2 * _F), lambda i: (i, 0)),
          pl.BlockSpec((_F, _F), lambda i: (0, 0)),
          pl.BlockSpec((1, _F), lambda i: (0, 0)),
          pl.BlockSpec((2 * _F, _FH), lambda i: (0, 0)),
      ],
      out_specs=pl.BlockSpec((bm, _F), lambda i: (i, 0)),
      out_shape=jax.ShapeDtypeStruct((_N1, _F), jnp.float32),
  )(sums, cnt, agg0, hist1, w1, b1.reshape(1, _F), w2)


@functools.partial(
    pl.kernel,
    out_type=jax.ShapeDtypeStruct((2, _ACC1, _F), jnp.float32),
    mesh=_mesh,
    scratch_types=[
        pltpu.VMEM((_EPT1,), jnp.int32),
        pltpu.VMEM((_NCH1, _K), jnp.int32),
        pltpu.VMEM((2, _K, _F), jnp.float32),
        pltpu.VMEM_SHARED((_ACC1, _F), jnp.float32),
        pltpu.SemaphoreType.DMA,
        pltpu.SemaphoreType.DMA,
        pltpu.SemaphoreType.DMA,
        pltpu.SemaphoreType.DMA,
    ],
)
def _agg1(y_r, srcs_r, dsts_r, z128_r, psum_r, src_v, dst_v, rows_v, acc,
          ga, gb, sa, sb):
  c = lax.axis_index("c")
  s = lax.axis_index("s")
  z0 = jnp.where(s == 15, _ACC1 - _ZSP1, s * _ZSP1)
  pltpu.sync_copy(z128_r.at[pl.ds(0, _ZSP1)], acc.at[pl.ds(z0, _ZSP1)])
  pltpu.sync_copy(srcs_r.at[c, pl.ds(s * _EPT1, _EPT1)], src_v)
  pltpu.sync_copy(dsts_r.at[c, s], dst_v)
  plsc.subcore_barrier()

  gsem = (ga, gb)
  ssem = (sa, sb)

  def gather(j, b):
    return pltpu.make_async_copy(
        y_r.at[src_v.at[pl.ds(j * _K, _K)]], rows_v.at[b], gsem[b])

  def scatter(j, b):
    return pltpu.make_async_copy(rows_v.at[b], acc.at[dst_v.at[j]], ssem[b])

  for j in range(_NCH1):
    b = j & 1
    if j >= 2:
      scatter(j - 2, b).wait()
    gj = gather(j, b)
    gj.start()
    gj.wait()
    scatter(j, b).start(add=True)
  scatter(_NCH1 - 2, 0).wait()
  scatter(_NCH1 - 1, 1).wait()
  plsc.subcore_barrier()
  pltpu.sync_copy(acc.at[pl.ds(z0, _ZSP1)], psum_r.at[c, pl.ds(z0, _ZSP1)])


def _post(psums, cnt1, agg1, w2, b2):
  def body(ps_r, c_r, a_r, w_r, b_r, o_r):
    sums = (ps_r[0] + ps_r[1])[:_N2, :_FH]
    n = jnp.maximum(jnp.sum(c_r[...], axis=1, keepdims=True), 1.0)
    z = jnp.dot(a_r[...], w_r[...], preferred_element_type=jnp.float32)
    o_r[...] = sums / n + z + b_r[...]

  return pl.pallas_call(
      body,
      grid=(1,),
      in_specs=[
          pl.BlockSpec((2, _ACC1, _F), lambda i: (0, 0, 0)),
          pl.BlockSpec((_N2, 32), lambda i: (0, 0)),
          pl.BlockSpec((_N2, 2 * _F), lambda i: (0, 0)),
          pl.BlockSpec((2 * _F, _FH), lambda i: (0, 0)),
          pl.BlockSpec((1, _FH), lambda i: (0, 0)),
      ],
      out_specs=pl.BlockSpec((_N2, _FH), lambda i: (0, 0)),
      out_shape=jax.ShapeDtypeStruct((_N2, _FH), jnp.float32),
  )(psums, cnt1, agg1, w2, b2.reshape(1, _FH))


def _pad1d(a, n, val):
  return jnp.concatenate([a, jnp.full((n - a.shape[0],), val, a.dtype)])


def kernel(preprocess, h_hist_0, h_hist_1, agg_h_0, agg_h_1,
           edge_index_0, edge_index_1, W0, b0, W1, b1, W2, b2):
  z128 = jnp.zeros((_ZSP0, _F), jnp.float32)
  z1d = jnp.zeros((_CNT0,), jnp.float32)

  src0 = _pad1d(edge_index_0[0], _E0P, 0)
  d0 = _pad1d(edge_index_0[1], _E0P, _N1)
  dsts0 = jnp.stack([
      jnp.where(d0 < _B0, d0, _B0),
      jnp.where(d0 >= _B0, d0 - _B0, _B0),
  ]).reshape(2, 16, _NCH0, _K0)
  half = _E1 // 2
  srcs1 = jnp.stack([
      _pad1d(edge_index_1[0, :half], _E1H, 0),
      _pad1d(edge_index_1[0, half:], _E1H, 0),
  ])
  dsts1 = jnp.stack([
      _pad1d(edge_index_1[1, :half], _E1H, _N2),
      _pad1d(edge_index_1[1, half:], _E1H, _N2),
  ]).reshape(2, 16, _NCH1, _K)

  cnt0, cnt1 = _counts(dsts0, dsts1, z1d)
  cnt0_t = jnp.concatenate(
      [cnt0[0].T[:_B0], cnt0[1].T[:_N1 - _B0]], axis=0)
  cnt1_t = jnp.concatenate([cnt1[0].T[:_N2], cnt1[1].T[:_N2]], axis=1)

  h = _pre(preprocess, h_hist_0, W0, b0)

  sums0 = _agg0(h, src0, dsts0, z128)

  y = _mid(sums0, cnt0_t, agg_h_0, h_hist_1, W1, b1, W2)

  psums = _agg1(y, srcs1, dsts1, z128)

  return _post(psums, cnt1_t, agg_h_1, W2, b2)


# speedup vs baseline: 1.4938x; 1.0191x over previous
import functools

import jax
import jax.numpy as jnp
from jax import lax
from jax.experimental import pallas as pl
from jax.experimental.pallas import tpu as pltpu
from jax.experimental.pallas import tpu_sc as plsc

_N0, _N1, _N2 = 100000, 25000, 2500
_E0, _E1 = 400000, 40000
_F = 128
_FH = 64
_K = 128

_K0 = 112
_G0 = 8
_NG0 = 28
_NCH0 = _NG0 * _G0
_EPT0 = _NCH0 * _K0
_E0P = 16 * _EPT0
_B0 = 12504
_ACC0 = 12512
_CNT0 = 12544
_ZSP0 = 784
_GSZ = _G0 * _K0

_NCH1 = 10
_EPT1 = _NCH1 * _K
_E1H = 16 * _EPT1
_ACC1 = 2512
_CNT1 = 2560
_ZSP1 = 160

_mesh = plsc.VectorSubcoreMesh(
    core_axis_name="c", subcore_axis_name="s", num_cores=2, num_subcores=16)


@functools.partial(
    pl.kernel,
    out_type=[
        jax.ShapeDtypeStruct((2, 16, _CNT0), jnp.float32),
        jax.ShapeDtypeStruct((2, 16, _CNT1), jnp.float32),
    ],
    mesh=_mesh,
    compiler_params=pltpu.CompilerParams(needs_layout_passes=False),
    scratch_types=[
        pltpu.VMEM((_NCH0, _K0), jnp.int32),
        pltpu.VMEM((_NCH1, _K), jnp.int32),
        pltpu.VMEM((_CNT0,), jnp.float32),
        pltpu.VMEM((_CNT1,), jnp.float32),
    ],
)
def _counts(d0_r, d1_r, z_r, c0_r, c1_r, d0_v, d1_v, c0_v, c1_v):
  c = lax.axis_index("c")
  s = lax.axis_index("s")
  pltpu.sync_copy(d0_r.at[c, s], d0_v)
  pltpu.sync_copy(d1_r.at[c, s], d1_v)
  pltpu.sync_copy(z_r, c0_v)
  pltpu.sync_copy(z_r.at[pl.ds(0, _CNT1)], c1_v)
  one16 = jnp.full((16,), 1.0, jnp.float32)

  def h0(j, carry):
    def inner(t, carry2):
      plsc.addupdate_scatter(c0_v, [d0_v[j, pl.ds(t * 16, 16)]], one16)
      return carry2
    return lax.fori_loop(0, _K0 // 16, inner, carry)

  lax.fori_loop(0, _NCH0, h0, 0)

  def h1(j, carry):
    def inner(t, carry2):
      plsc.addupdate_scatter(c1_v, [d1_v[j, pl.ds(t * 16, 16)]], one16)
      return carry2
    return lax.fori_loop(0, _K // 16, inner, carry)

  lax.fori_loop(0, _NCH1, h1, 0)
  pltpu.sync_copy(c0_v, c0_r.at[c, s])
  pltpu.sync_copy(c1_v, c1_r.at[c, s])


def _pre(x, hist, w0, b0):
  bm = 1000

  def body(x_r, h_r, w_r, b_r, o_r):
    h = jnp.dot(x_r[...], w_r[...], preferred_element_type=jnp.float32)
    o_r[...] = jnp.maximum(h + b_r[...], 0.0) - h_r[...]

  return pl.pallas_call(
      body,
      grid=(_N0 // bm,),
      in_specs=[
          pl.BlockSpec((bm, _F), lambda i: (i, 0)),
          pl.BlockSpec((bm, _F), lambda i: (i, 0)),
          pl.BlockSpec((_F, _F), lambda i: (0, 0)),
          pl.BlockSpec((1, _F), lambda i: (0, 0)),
      ],
      out_specs=pl.BlockSpec((bm, _F), lambda i: (i, 0)),
      out_shape=jax.ShapeDtypeStruct((_N0, _F), jnp.float32),
  )(x, hist, w0, b0.reshape(1, _F))


@functools.partial(
    pl.kernel,
    out_type=jax.ShapeDtypeStruct((_N1, _F), jnp.float32),
    mesh=_mesh,
    scratch_types=[
        pltpu.VMEM((_GSZ,), jnp.int32),
        pltpu.VMEM((_G0, _K0), jnp.int32),
        pltpu.VMEM((2, _K0, _F), jnp.float32),
        pltpu.VMEM_SHARED((_ACC0, _F), jnp.float32),
        pltpu.SemaphoreType.DMA,
        pltpu.SemaphoreType.DMA,
        pltpu.SemaphoreType.DMA,
        pltpu.SemaphoreType.DMA,
    ],
)
def _agg0(h_r, src_r, dsts_r, z128_r, sums_r,
          src_g, dst_g, rows_v, acc, ga, gb, sa, sb):
  c = lax.axis_index("c")
  s = lax.axis_index("s")
  z0 = jnp.where(s == 15, _ACC0 - _ZSP0, s * _ZSP0)
  pltpu.sync_copy(z128_r, acc.at[pl.ds(z0, _ZSP0)])
  plsc.subcore_barrier()

  gsem = (ga, gb)
  ssem = (sa, sb)

  def group(g, carry):
    pltpu.sync_copy(src_r.at[pl.ds(s * _EPT0 + g * _GSZ, _GSZ)], src_g)
    pltpu.sync_copy(dsts_r.at[c, s, pl.ds(g * _G0, _G0)], dst_g)

    def gather(k, b):
      return pltpu.make_async_copy(
          h_r.at[src_g.at[pl.ds(k * _K0, _K0)]], rows_v.at[b], gsem[b])

    def scatter(k, b):
      return pltpu.make_async_copy(rows_v.at[b], acc.at[dst_g.at[k]], ssem[b])

    gather(0, 0).start()
    for k in range(_G0):
      b = k & 1
      if k >= 1:
        scatter(k - 1, 1 - b).wait()
      if k + 1 < _G0:
        gather(k + 1, 1 - b).start()
      gather(k, b).wait()
      scatter(k, b).start(add=True)
    scatter(_G0 - 1, 1).wait()
    return carry

  lax.fori_loop(0, _NG0, group, 0)
  plsc.subcore_barrier()
  w0 = jnp.where(s == 15,
                 jnp.where(c == 0, _B0 - _ZSP0, (_N1 - _B0) - _ZSP0),
                 s * _ZSP0)
  pltpu.sync_copy(acc.at[pl.ds(w0, _ZSP0)],
                  sums_r.at[pl.ds(c * _B0 + w0, _ZSP0)])


def _mid(sums, cnt, agg0, hist1, w1, b1, w2):
  bm = 1000

  def body(s_r, c_r, a_r, h_r, w1_r, b1_r, w2_r, y_r):
    n = jnp.maximum(jnp.sum(c_r[...], axis=1, keepdims=True), 1.0)
    m = s_r[...] / n + a_r[...]
    t = jnp.dot(m, w1_r[...], preferred_element_type=jnp.float32) + b1_r[...]
    tcat = jnp.concatenate([t, jnp.maximum(t, 0.0)], axis=1) - h_r[...]
    y = jnp.dot(tcat, w2_r[...], preferred_element_type=jnp.float32)
    y_r[...] = jnp.concatenate(
        [y, jnp.zeros((bm, _F - _FH), jnp.float32)], axis=1)

  return pl.pallas_call(
      body,
      grid=(_N1 // bm,),
      in_specs=[
          pl.BlockSpec((bm, _F), lambda i: (i, 0)),
          pl.BlockSpec((bm, 16), lambda i: (i, 0)),
          pl.BlockSpec((bm, _F), lambda i: (i, 0)),
          pl.BlockSpec((bm, 2 * _F), lambda i: (i, 0)),
          pl.BlockSpec((_F, _F), lambda i: (0, 0)),
          pl.BlockSpec((1, _F), lambda i: (0, 0)),
          pl.BlockSpec((2 * _F, _FH), lambda i: (0, 0)),
      ],
      out_specs=pl.BlockSpec((bm, _F), lambda i: (i, 0)),
      out_shape=jax.ShapeDtypeStruct((_N1, _F), jnp.float32),
  )(sums, cnt, agg0, hist1, w1, b1.reshape(1, _F), w2)


@functools.partial(
    pl.kernel,
    out_type=jax.ShapeDtypeStruct((2, _ACC1, _F), jnp.float32),
    mesh=_mesh,
    scratch_types=[
        pltpu.VMEM((_EPT1,), jnp.int32),
        pltpu.VMEM((_NCH1, _K), jnp.int32),
        pltpu.VMEM((2, _K, _F), jnp.float32),
        pltpu.VMEM_SHARED((_ACC1, _F), jnp.float32),
        pltpu.SemaphoreType.DMA,
        pltpu.SemaphoreType.DMA,
        pltpu.SemaphoreType.DMA,
        pltpu.SemaphoreType.DMA,
    ],
)
def _agg1(y_r, srcs_r, dsts_r, z128_r, psum_r, src_v, dst_v, rows_v, acc,
          ga, gb, sa, sb):
  c = lax.axis_index("c")
  s = lax.axis_index("s")
  z0 = jnp.where(s == 15, _ACC1 - _ZSP1, s * _ZSP1)
  pltpu.sync_copy(z128_r.at[pl.ds(0, _ZSP1)], acc.at[pl.ds(z0, _ZSP1)])
  pltpu.sync_copy(srcs_r.at[c, pl.ds(s * _EPT1, _EPT1)], src_v)
  pltpu.sync_copy(dsts_r.at[c, s], dst_v)
  plsc.subcore_barrier()

  gsem = (ga, gb)
  ssem = (sa, sb)

  def gather(j, b):
    return pltpu.make_async_copy(
        y_r.at[src_v.at[pl.ds(j * _K, _K)]], rows_v.at[b], gsem[b])

  def scatter(j, b):
    return pltpu.make_async_copy(rows_v.at[b], acc.at[dst_v.at[j]], ssem[b])

  for j in range(_NCH1):
    b = j & 1
    if j >= 2:
      scatter(j - 2, b).wait()
    gj = gather(j, b)
    gj.start()
    gj.wait()
    scatter(j, b).start(add=True)
  scatter(_NCH1 - 2, 0).wait()
  scatter(_NCH1 - 1, 1).wait()
  plsc.subcore_barrier()
  pltpu.sync_copy(acc.at[pl.ds(z0, _ZSP1)], psum_r.at[c, pl.ds(z0, _ZSP1)])


def _post(psums, cnt1, agg1, w2, b2):
  def body(ps_r, c_r, a_r, w_r, b_r, o_r):
    sums = (ps_r[0] + ps_r[1])[:_N2, :_FH]
    n = jnp.maximum(jnp.sum(c_r[...], axis=1, keepdims=True), 1.0)
    z = jnp.dot(a_r[...], w_r[...], preferred_element_type=jnp.float32)
    o_r[...] = sums / n + z + b_r[...]

  return pl.pallas_call(
      body,
      grid=(1,),
      in_specs=[
          pl.BlockSpec((2, _ACC1, _F), lambda i: (0, 0, 0)),
          pl.BlockSpec((_N2, 32), lambda i: (0, 0)),
          pl.BlockSpec((_N2, 2 * _F), lambda i: (0, 0)),
          pl.BlockSpec((2 * _F, _FH), lambda i: (0, 0)),
          pl.BlockSpec((1, _FH), lambda i: (0, 0)),
      ],
      out_specs=pl.BlockSpec((_N2, _FH), lambda i: (0, 0)),
      out_shape=jax.ShapeDtypeStruct((_N2, _FH), jnp.float32),
  )(psums, cnt1, agg1, w2, b2.reshape(1, _FH))


def _pad1d(a, n, val):
  return jnp.concatenate([a, jnp.full((n - a.shape[0],), val, a.dtype)])


def kernel(preprocess, h_hist_0, h_hist_1, agg_h_0, agg_h_1,
           edge_index_0, edge_index_1, W0, b0, W1, b1, W2, b2):
  z128 = jnp.zeros((_ZSP0, _F), jnp.float32)
  z1d = jnp.zeros((_CNT0,), jnp.float32)

  src0 = _pad1d(edge_index_0[0], _E0P, 0)
  d0 = _pad1d(edge_index_0[1], _E0P, _N1)
  dsts0 = jnp.stack([
      jnp.where(d0 < _B0, d0, _B0),
      jnp.where(d0 >= _B0, d0 - _B0, _B0),
  ]).reshape(2, 16, _NCH0, _K0)
  half = _E1 // 2
  srcs1 = jnp.stack([
      _pad1d(edge_index_1[0, :half], _E1H, 0),
      _pad1d(edge_index_1[0, half:], _E1H, 0),
  ])
  dsts1 = jnp.stack([
      _pad1d(edge_index_1[1, :half], _E1H, _N2),
      _pad1d(edge_index_1[1, half:], _E1H, _N2),
  ]).reshape(2, 16, _NCH1, _K)

  cnt0, cnt1 = _counts(dsts0, dsts1, z1d)
  cnt0_t = jnp.concatenate(
      [cnt0[0].T[:_B0], cnt0[1].T[:_N1 - _B0]], axis=0)
  cnt1_t = jnp.concatenate([cnt1[0].T[:_N2], cnt1[1].T[:_N2]], axis=1)

  h = _pre(preprocess, h_hist_0, W0, b0)

  sums0 = _agg0(h, src0, dsts0, z128)

  y = _mid(sums0, cnt0_t, agg_h_0, h_hist_1, W1, b1, W2)

  psums = _agg1(y, srcs1, dsts1, z128)

  return _post(psums, cnt1_t, agg_h_1, W2, b2)
